# baseline (speedup 1.0000x reference)
import functools

import jax
import jax.numpy as jnp
from jax import lax
from jax.experimental import pallas as pl
from jax.experimental.pallas import tpu as pltpu
from jax.experimental.pallas import tpu_sc as plsc

N, E, D, ED, H, L = 10000, 320000, 128, 16, 128, 3
LANES = 16
NC, NS = 2, 16
NPAD = 10240
NW = NC * NS
EPT = E // NW
C = 80
ITERS = EPT // C


def _proj_body(x_ref, w_ref, b_ref, o_ref):
    o_ref[...] = (
        jax.lax.dot_general(x_ref[...], w_ref[...], (((1,), (0,)), ((), ())),
                            preferred_element_type=jnp.float32,
                            precision=jax.lax.Precision.HIGHEST)
        + b_ref[...]
    )


def _proj(x, w, b, blk):
    n = x.shape[0]
    return pl.pallas_call(
        _proj_body,
        grid=(n // blk,),
        in_specs=[
            pl.BlockSpec((blk, x.shape[1]), lambda i: (i, jnp.int32(0))),
            pl.BlockSpec(w.shape, lambda i: (jnp.int32(0), jnp.int32(0))),
            pl.BlockSpec((1, b.shape[1]), lambda i: (jnp.int32(0), jnp.int32(0))),
        ],
        out_specs=pl.BlockSpec((blk, w.shape[1]), lambda i: (i, jnp.int32(0))),
        out_shape=jax.ShapeDtypeStruct((n, w.shape[1]), jnp.float32),
    )(x, w, b)


def _node_body(h_ref, a0_ref, a1_ref, scale_ref, w1_ref, b1_ref, w2_ref,
               b2_ref, g_ref, be_ref, o_ref):
    z = scale_ref[0, 0] * h_ref[...] + a0_ref[...] + a1_ref[...]
    t = jax.lax.dot_general(z, w1_ref[...], (((1,), (0,)), ((), ())),
                            preferred_element_type=jnp.float32,
                            precision=jax.lax.Precision.HIGHEST)
    t = jnp.maximum(t + b1_ref[...], 0.0)
    y = jax.lax.dot_general(t, w2_ref[...], (((1,), (0,)), ((), ())),
                            preferred_element_type=jnp.float32,
                            precision=jax.lax.Precision.HIGHEST)
    y = y + b2_ref[...]
    mu = jnp.mean(y, axis=-1, keepdims=True)
    var = jnp.mean((y - mu) ** 2, axis=-1, keepdims=True)
    y = (y - mu) * jax.lax.rsqrt(var + 1e-5) * g_ref[...] + be_ref[...]
    o_ref[...] = jnp.maximum(y, 0.0)


def _node_update(h, a0, a1, scale, w1, b1, w2, b2, gamma, beta, blk):
    n = h.shape[0]
    return pl.pallas_call(
        _node_body,
        grid=(n // blk,),
        in_specs=[
            pl.BlockSpec((blk, H), lambda i: (i, jnp.int32(0))),
            pl.BlockSpec((blk, H), lambda i: (i, jnp.int32(0))),
            pl.BlockSpec((blk, H), lambda i: (i, jnp.int32(0))),
            pl.BlockSpec((1, 1), lambda i: (jnp.int32(0), jnp.int32(0)),
                         memory_space=pltpu.SMEM),
            pl.BlockSpec((H, 2 * H), lambda i: (jnp.int32(0), jnp.int32(0))),
            pl.BlockSpec((1, 2 * H), lambda i: (jnp.int32(0), jnp.int32(0))),
            pl.BlockSpec((2 * H, H), lambda i: (jnp.int32(0), jnp.int32(0))),
            pl.BlockSpec((1, H), lambda i: (jnp.int32(0), jnp.int32(0))),
            pl.BlockSpec((1, H), lambda i: (jnp.int32(0), jnp.int32(0))),
            pl.BlockSpec((1, H), lambda i: (jnp.int32(0), jnp.int32(0))),
        ],
        out_specs=pl.BlockSpec((blk, H), lambda i: (i, jnp.int32(0))),
        out_shape=jax.ShapeDtypeStruct((n, H), jnp.float32),
    )(h, a0, a1, scale, w1, b1, w2, b2, gamma, beta)


CH = 64
ROWS = E // CH
PER = 12
KCH = -(-ROWS // NW)
KCH = KCH + (-KCH % PER)
ROWS_PAD = NW * (KCH + 3)
TRASH = N


@functools.cache
def _build_agg():
    return functools.partial(
        pl.kernel,
        out_type=jax.ShapeDtypeStruct((NC, NPAD, H), jnp.float32),
        mesh=plsc.VectorSubcoreMesh(core_axis_name="c", subcore_axis_name="s",
                                    num_cores=NC, num_subcores=NS),
        scratch_types=[
            pltpu.VMEM((4, 2, CH), jnp.int32),
            pltpu.VMEM((2, CH, H), jnp.float32),
            pltpu.VMEM((3, CH, H), jnp.float32),
            pltpu.VMEM_SHARED((NPAD, H), jnp.float32),
            pltpu.SemaphoreType.DMA,
            pltpu.SemaphoreType.DMA,
            pltpu.SemaphoreType.DMA,
            pltpu.SemaphoreType.DMA,
            pltpu.SemaphoreType.DMA,
            pltpu.SemaphoreType.DMA,
            pltpu.SemaphoreType.DMA,
            pltpu.SemaphoreType.DMA,
            pltpu.SemaphoreType.DMA,
        ],
    )(_agg_body)


def _agg_body(e_hbm, h_hbm, sd_hbm, zeros_hbm, out_hbm,
              idx4, ebuf, gbuf, agg_sh,
              si0, si1, se0, se1, sg0, sg1, sg2, sc0, sc1):
    c = lax.axis_index("c")
    s = lax.axis_index("s")
    wid = s * NC + c
    sis, ses, sgs, scs = (si0, si1), (se0, se1), (sg0, sg1, sg2), (sc0, sc1)
    emax = jnp.int32(E - CH)

    pltpu.sync_copy(zeros_hbm, agg_sh.at[pl.ds(s * (NPAD // NS), NPAD // NS)])
    plsc.subcore_barrier()

    def row_of(x):
        return wid + x * jnp.int32(NW)

    def i_idx(x, j):
        pltpu.async_copy(sd_hbm.at[row_of(x)], idx4.at[jnp.int32(j & 3)],
                         sis[j & 1])

    def w_idx(j):
        pltpu.make_async_copy(sd_hbm.at[jnp.int32(0)],
                              idx4.at[jnp.int32(j & 3)], sis[j & 1]).wait()

    def i_e(x, p):
        base = jnp.minimum(row_of(x) * jnp.int32(CH), emax)
        pltpu.async_copy(e_hbm.at[pl.ds(base, CH)],
                         ebuf.at[jnp.int32(p & 1)], ses[p & 1])

    def w_e(p):
        pltpu.make_async_copy(e_hbm.at[pl.ds(0, CH)],
                              ebuf.at[jnp.int32(p & 1)], ses[p & 1]).wait()

    def i_g(x, j, r):
        pltpu.async_copy(h_hbm.at[idx4.at[jnp.int32(j & 3), jnp.int32(0)]],
                         gbuf.at[jnp.int32(r % 3)], sgs[r % 3])

    def w_g(r):
        pltpu.make_async_copy(e_hbm.at[pl.ds(0, CH)],
                              gbuf.at[jnp.int32(r % 3)], sgs[r % 3]).wait()

    def i_sc(j, r, p):
        pltpu.async_copy(gbuf.at[jnp.int32(r % 3)],
                         agg_sh.at[idx4.at[jnp.int32(j & 3), jnp.int32(1)]],
                         scs[p & 1], add=True)

    def w_sc(p):
        pltpu.make_async_copy(gbuf.at[jnp.int32(0)],
                              agg_sh.at[pl.ds(0, CH)], scs[p & 1]).wait()

    def compute(r, p):
        gb, eb = gbuf.at[jnp.int32(r % 3)], ebuf.at[jnp.int32(p & 1)]

        def rowfn(i, cr):
            for jj in range(H // LANES):
                sl = pl.ds(jj * LANES, LANES)
                gb[i, sl] = jnp.maximum(gb[i, sl] + eb[i, sl], 0.0)
            return cr

        lax.fori_loop(jnp.int32(0), jnp.int32(CH), rowfn, jnp.int32(0))

    def process(x, m, skip_sc=False, skip_idx3=False):
        w_e(m)
        w_g(m)
        compute(m, m)
        i_sc(m, m, m)
        i_e(x + 2, m)
        if not skip_sc:
            w_sc(m + 1)
        if not skip_idx3:
            i_idx(x + 3, m + 3)
        w_idx(m + 2)
        i_g(x + 2, m + 2, m + 2)

    i_idx(jnp.int32(0), 0)
    i_idx(jnp.int32(1), 1)
    w_idx(0)
    i_idx(jnp.int32(2), 2)
    i_e(jnp.int32(0), 0)
    i_e(jnp.int32(1), 1)
    i_g(jnp.int32(0), 0, 0)
    w_idx(1)
    i_idx(jnp.int32(3), 3)
    i_g(jnp.int32(1), 1, 1)

    process(jnp.int32(0), 0, skip_sc=True, skip_idx3=True)
    for m in range(1, PER):
        process(jnp.int32(m), m)

    def body(k, cr):
        x = k * jnp.int32(PER)
        for m in range(PER):
            process(x + m, m)
        return cr

    lax.fori_loop(jnp.int32(1), jnp.int32(KCH // PER), body, jnp.int32(0))

    w_sc(KCH - 1)
    w_e(0)
    w_e(1)
    w_idx(KCH + 2)
    w_g(KCH)
    w_g(KCH + 1)
    plsc.subcore_barrier()

    row0 = s * (NPAD // NS)
    pltpu.sync_copy(agg_sh.at[pl.ds(row0, NPAD // NS)],
                    out_hbm.at[c, pl.ds(row0, NPAD // NS)])


def kernel(x, ei, ea, W_proj, b_proj, eps, We, be, Wm1, bm1, Wm2, bm2,
           gamma, beta):
    src = ei[0].astype(jnp.int32).reshape(ROWS, CH)
    dst = ei[1].astype(jnp.int32).reshape(ROWS, CH)
    sd = jnp.stack([src, dst], axis=1)
    pad = jnp.full((ROWS_PAD - ROWS, 2, CH), TRASH, jnp.int32)
    pad = pad.at[:, 0, :].set(0)
    sd = jnp.concatenate([sd, pad], axis=0)
    zeros = jnp.zeros((NPAD // NS, H), jnp.float32)

    h = _proj(x, W_proj, b_proj.reshape(1, H), 1000)

    for l in range(L):
        e = _proj(ea, We[l], be[l].reshape(1, H), 4000)
        aggp = _build_agg()(e, h, sd, zeros)
        scale = (1.0 + eps[l]).reshape(1, 1).astype(jnp.float32)
        h = _node_update(h, aggp[0], aggp[1], scale, Wm1[l],
                         bm1[l].reshape(1, 2 * H), Wm2[l],
                         bm2[l].reshape(1, H), gamma[l].reshape(1, H),
                         beta[l].reshape(1, H), 1000)
    return h


# speedup vs baseline: 1.8949x; 1.8949x over previous
import functools

import jax
import jax.numpy as jnp
from jax import lax
from jax.experimental import pallas as pl
from jax.experimental.pallas import tpu as pltpu
from jax.experimental.pallas import tpu_sc as plsc

N, E, D, ED, H, L = 10000, 320000, 128, 16, 128, 3
LANES = 16
NC, NS = 2, 16
NPAD = 10240
NW = NC * NS
EPT = E // NW
C = 80
ITERS = EPT // C


def _proj_body(x_ref, w_ref, b_ref, o_ref):
    o_ref[...] = (
        jax.lax.dot_general(x_ref[...], w_ref[...], (((1,), (0,)), ((), ())),
                            preferred_element_type=jnp.float32,
                            precision=jax.lax.Precision.HIGHEST)
        + b_ref[...]
    )


def _proj(x, w, b, blk):
    n = x.shape[0]
    return pl.pallas_call(
        _proj_body,
        grid=(n // blk,),
        in_specs=[
            pl.BlockSpec((blk, x.shape[1]), lambda i: (i, jnp.int32(0))),
            pl.BlockSpec(w.shape, lambda i: (jnp.int32(0), jnp.int32(0))),
            pl.BlockSpec((1, b.shape[1]), lambda i: (jnp.int32(0), jnp.int32(0))),
        ],
        out_specs=pl.BlockSpec((blk, w.shape[1]), lambda i: (i, jnp.int32(0))),
        out_shape=jax.ShapeDtypeStruct((n, w.shape[1]), jnp.float32),
    )(x, w, b)


def _node_body(h_ref, a0_ref, a1_ref, scale_ref, w1_ref, b1_ref, w2_ref,
               b2_ref, g_ref, be_ref, o_ref):
    z = scale_ref[0, 0] * h_ref[...] + a0_ref[...] + a1_ref[...]
    t = jax.lax.dot_general(z, w1_ref[...], (((1,), (0,)), ((), ())),
                            preferred_element_type=jnp.float32,
                            precision=jax.lax.Precision.HIGHEST)
    t = jnp.maximum(t + b1_ref[...], 0.0)
    y = jax.lax.dot_general(t, w2_ref[...], (((1,), (0,)), ((), ())),
                            preferred_element_type=jnp.float32,
                            precision=jax.lax.Precision.HIGHEST)
    y = y + b2_ref[...]
    mu = jnp.mean(y, axis=-1, keepdims=True)
    var = jnp.mean((y - mu) ** 2, axis=-1, keepdims=True)
    y = (y - mu) * jax.lax.rsqrt(var + 1e-5) * g_ref[...] + be_ref[...]
    o_ref[...] = jnp.maximum(y, 0.0)


def _node_update(h, a0, a1, scale, w1, b1, w2, b2, gamma, beta, blk):
    n = h.shape[0]
    return pl.pallas_call(
        _node_body,
        grid=(n // blk,),
        in_specs=[
            pl.BlockSpec((blk, H), lambda i: (i, jnp.int32(0))),
            pl.BlockSpec((blk, H), lambda i: (i, jnp.int32(0))),
            pl.BlockSpec((blk, H), lambda i: (i, jnp.int32(0))),
            pl.BlockSpec((1, 1), lambda i: (jnp.int32(0), jnp.int32(0)),
                         memory_space=pltpu.SMEM),
            pl.BlockSpec((H, 2 * H), lambda i: (jnp.int32(0), jnp.int32(0))),
            pl.BlockSpec((1, 2 * H), lambda i: (jnp.int32(0), jnp.int32(0))),
            pl.BlockSpec((2 * H, H), lambda i: (jnp.int32(0), jnp.int32(0))),
            pl.BlockSpec((1, H), lambda i: (jnp.int32(0), jnp.int32(0))),
            pl.BlockSpec((1, H), lambda i: (jnp.int32(0), jnp.int32(0))),
            pl.BlockSpec((1, H), lambda i: (jnp.int32(0), jnp.int32(0))),
        ],
        out_specs=pl.BlockSpec((blk, H), lambda i: (i, jnp.int32(0))),
        out_shape=jax.ShapeDtypeStruct((n, H), jnp.float32),
    )(h, a0, a1, scale, w1, b1, w2, b2, gamma, beta)


CH = 80
ROWS = E // CH
PER = 4
KCH = -(-ROWS // NW)
KCH = KCH + (-KCH % PER)
ROWS_PAD = NW * (KCH + 2)
TRASH = N


@functools.cache
def _build_agg():
    return functools.partial(
        pl.kernel,
        out_type=jax.ShapeDtypeStruct((NC, NPAD, H), jnp.float32),
        mesh=plsc.VectorSubcoreMesh(core_axis_name="c", subcore_axis_name="s",
                                    num_cores=NC, num_subcores=NS),
        scratch_types=[
            pltpu.VMEM((4, 2, CH), jnp.int32),
            pltpu.VMEM((2, CH, H), jnp.float32),
            pltpu.VMEM((2, CH, H), jnp.float32),
            pltpu.VMEM_SHARED((NPAD, H), jnp.float32),
            pltpu.SemaphoreType.DMA,
            pltpu.SemaphoreType.DMA,
            pltpu.SemaphoreType.DMA,
            pltpu.SemaphoreType.DMA,
            pltpu.SemaphoreType.DMA,
            pltpu.SemaphoreType.DMA,
            pltpu.SemaphoreType.DMA,
            pltpu.SemaphoreType.DMA,
        ],
    )(_agg_body)


def _agg_body(e_hbm, h_hbm, sd_hbm, zeros_hbm, out_hbm,
              idx4, ebuf, gbuf, agg_sh,
              si0, si1, se0, se1, sg0, sg1, sc0, sc1):
    c = lax.axis_index("c")
    s = lax.axis_index("s")
    wid = s * NC + c
    sis, ses, sgs, scs = (si0, si1), (se0, se1), (sg0, sg1), (sc0, sc1)
    emax = jnp.int32(E - CH)

    pltpu.sync_copy(zeros_hbm, agg_sh.at[pl.ds(s * (NPAD // NS), NPAD // NS)])
    plsc.subcore_barrier()

    def row_of(x):
        return wid + x * jnp.int32(NW)

    def i_idx(x, j):
        pltpu.async_copy(sd_hbm.at[row_of(x)], idx4.at[jnp.int32(j & 3)],
                         sis[j & 1])

    def w_idx(j):
        pltpu.make_async_copy(sd_hbm.at[jnp.int32(0)],
                              idx4.at[jnp.int32(j & 3)], sis[j & 1]).wait()

    def i_e(x, p):
        base = jnp.minimum(row_of(x) * jnp.int32(CH), emax)
        pltpu.async_copy(e_hbm.at[pl.ds(base, CH)],
                         ebuf.at[jnp.int32(p & 1)], ses[p & 1])

    def w_e(p):
        pltpu.make_async_copy(e_hbm.at[pl.ds(0, CH)],
                              ebuf.at[jnp.int32(p & 1)], ses[p & 1]).wait()

    def i_g(x, j, p):
        pltpu.async_copy(h_hbm.at[idx4.at[jnp.int32(j & 3), jnp.int32(0)]],
                         gbuf.at[jnp.int32(p & 1)], sgs[p & 1])

    def w_g(p):
        pltpu.make_async_copy(e_hbm.at[pl.ds(0, CH)],
                              gbuf.at[jnp.int32(p & 1)], sgs[p & 1]).wait()

    def i_sc(j, p):
        pltpu.async_copy(gbuf.at[jnp.int32(p & 1)],
                         agg_sh.at[idx4.at[jnp.int32(j & 3), jnp.int32(1)]],
                         scs[p & 1], add=True)

    def w_sc(p):
        pltpu.make_async_copy(gbuf.at[jnp.int32(0)],
                              agg_sh.at[pl.ds(0, CH)], scs[p & 1]).wait()

    def compute(p):
        gb, eb = gbuf.at[jnp.int32(p & 1)], ebuf.at[jnp.int32(p & 1)]

        def rowfn(i, cr):
            for jj in range(H // LANES):
                sl = pl.ds(jj * LANES, LANES)
                gb[i, sl] = jnp.maximum(gb[i, sl] + eb[i, sl], 0.0)
            return cr

        lax.fori_loop(jnp.int32(0), jnp.int32(CH), rowfn, jnp.int32(0))

    def phase(x, m, first=False):
        p = m & 1
        q = 1 - p
        w_e(p)
        w_g(p)
        w_idx(m + 1)
        if not first:
            w_sc(q)
        i_g(x + 1, m + 1, q)
        compute(p)
        i_sc(m, p)
        i_e(x + 2, p)
        i_idx(x + 2, m + 2)

    i_idx(jnp.int32(0), 0)
    i_idx(jnp.int32(1), 1)
    i_e(jnp.int32(0), 0)
    i_e(jnp.int32(1), 1)
    w_idx(0)
    i_g(jnp.int32(0), 0, 0)

    phase(jnp.int32(0), 0, first=True)
    phase(jnp.int32(1), 1)
    phase(jnp.int32(2), 2)
    phase(jnp.int32(3), 3)

    def body(k, cr):
        x = k * jnp.int32(PER)
        for m in range(PER):
            phase(x + m, m)
        return cr

    lax.fori_loop(jnp.int32(1), jnp.int32(KCH // PER), body, jnp.int32(0))

    w_sc(1)
    w_e(0)
    w_e(1)
    w_idx(1)
    w_g(0)
    plsc.subcore_barrier()

    row0 = s * (NPAD // NS)
    pltpu.sync_copy(agg_sh.at[pl.ds(row0, NPAD // NS)],
                    out_hbm.at[c, pl.ds(row0, NPAD // NS)])


def kernel(x, ei, ea, W_proj, b_proj, eps, We, be, Wm1, bm1, Wm2, bm2,
           gamma, beta):
    src = ei[0].astype(jnp.int32).reshape(ROWS, CH)
    dst = ei[1].astype(jnp.int32).reshape(ROWS, CH)
    sd = jnp.stack([src, dst], axis=1)
    pad = jnp.full((ROWS_PAD - ROWS, 2, CH), TRASH, jnp.int32)
    pad = pad.at[:, 0, :].set(0)
    sd = jnp.concatenate([sd, pad], axis=0)
    zeros = jnp.zeros((NPAD // NS, H), jnp.float32)

    h = _proj(x, W_proj, b_proj.reshape(1, H), 1000)

    for l in range(L):
        e = _proj(ea, We[l], be[l].reshape(1, H), 4000)
        aggp = _build_agg()(e, h, sd, zeros)
        scale = (1.0 + eps[l]).reshape(1, 1).astype(jnp.float32)
        h = _node_update(h, aggp[0], aggp[1], scale, Wm1[l],
                         bm1[l].reshape(1, 2 * H), Wm2[l],
                         bm2[l].reshape(1, H), gamma[l].reshape(1, H),
                         beta[l].reshape(1, H), 1000)
    return h


# speedup vs baseline: 3.9382x; 2.0783x over previous
import functools

import jax
import jax.numpy as jnp
from jax import lax
from jax.experimental import pallas as pl
from jax.experimental.pallas import tpu as pltpu
from jax.experimental.pallas import tpu_sc as plsc

N, E, D, ED, H, L = 10000, 320000, 128, 16, 128, 3
LANES = 16
NC, NS = 2, 16
NPAD = 10240
NW = NC * NS
EPT = E // NW
C = 80
ITERS = EPT // C


def _proj_body(x_ref, w_ref, b_ref, o_ref):
    o_ref[...] = (
        jax.lax.dot_general(x_ref[...], w_ref[...], (((1,), (0,)), ((), ())),
                            preferred_element_type=jnp.float32,
                            precision=jax.lax.Precision.HIGHEST)
        + b_ref[...]
    )


def _proj(x, w, b, blk):
    n = x.shape[0]
    return pl.pallas_call(
        _proj_body,
        grid=(n // blk,),
        in_specs=[
            pl.BlockSpec((blk, x.shape[1]), lambda i: (i, jnp.int32(0))),
            pl.BlockSpec(w.shape, lambda i: (jnp.int32(0), jnp.int32(0))),
            pl.BlockSpec((1, b.shape[1]), lambda i: (jnp.int32(0), jnp.int32(0))),
        ],
        out_specs=pl.BlockSpec((blk, w.shape[1]), lambda i: (i, jnp.int32(0))),
        out_shape=jax.ShapeDtypeStruct((n, w.shape[1]), jnp.float32),
    )(x, w, b)


def _node_body(h_ref, a0_ref, a1_ref, scale_ref, w1_ref, b1_ref, w2_ref,
               b2_ref, g_ref, be_ref, o_ref):
    z = scale_ref[0, 0] * h_ref[...] + a0_ref[...] + a1_ref[...]
    t = jax.lax.dot_general(z, w1_ref[...], (((1,), (0,)), ((), ())),
                            preferred_element_type=jnp.float32,
                            precision=jax.lax.Precision.HIGHEST)
    t = jnp.maximum(t + b1_ref[...], 0.0)
    y = jax.lax.dot_general(t, w2_ref[...], (((1,), (0,)), ((), ())),
                            preferred_element_type=jnp.float32,
                            precision=jax.lax.Precision.HIGHEST)
    y = y + b2_ref[...]
    mu = jnp.mean(y, axis=-1, keepdims=True)
    var = jnp.mean((y - mu) ** 2, axis=-1, keepdims=True)
    y = (y - mu) * jax.lax.rsqrt(var + 1e-5) * g_ref[...] + be_ref[...]
    o_ref[...] = jnp.maximum(y, 0.0)


def _node_update(h, a0, a1, scale, w1, b1, w2, b2, gamma, beta, blk):
    n = h.shape[0]
    return pl.pallas_call(
        _node_body,
        grid=(n // blk,),
        in_specs=[
            pl.BlockSpec((blk, H), lambda i: (i, jnp.int32(0))),
            pl.BlockSpec((blk, H), lambda i: (i, jnp.int32(0))),
            pl.BlockSpec((blk, H), lambda i: (i, jnp.int32(0))),
            pl.BlockSpec((1, 1), lambda i: (jnp.int32(0), jnp.int32(0)),
                         memory_space=pltpu.SMEM),
            pl.BlockSpec((H, 2 * H), lambda i: (jnp.int32(0), jnp.int32(0))),
            pl.BlockSpec((1, 2 * H), lambda i: (jnp.int32(0), jnp.int32(0))),
            pl.BlockSpec((2 * H, H), lambda i: (jnp.int32(0), jnp.int32(0))),
            pl.BlockSpec((1, H), lambda i: (jnp.int32(0), jnp.int32(0))),
            pl.BlockSpec((1, H), lambda i: (jnp.int32(0), jnp.int32(0))),
            pl.BlockSpec((1, H), lambda i: (jnp.int32(0), jnp.int32(0))),
        ],
        out_specs=pl.BlockSpec((blk, H), lambda i: (i, jnp.int32(0))),
        out_shape=jax.ShapeDtypeStruct((n, H), jnp.float32),
    )(h, a0, a1, scale, w1, b1, w2, b2, gamma, beta)


C = 80
ITERS = EPT // C


@functools.cache
def _build_agg():
    return functools.partial(
        pl.kernel,
        out_type=jax.ShapeDtypeStruct((NC, NPAD, H), jnp.float32),
        mesh=plsc.VectorSubcoreMesh(core_axis_name="c", subcore_axis_name="s",
                                    num_cores=NC, num_subcores=NS),
        scratch_types=[
            pltpu.VMEM((C,), jnp.int32),
            pltpu.VMEM((C,), jnp.int32),
            pltpu.VMEM((C,), jnp.int32),
            pltpu.VMEM((C,), jnp.int32),
            pltpu.VMEM((C, H), jnp.float32),
            pltpu.VMEM((C, H), jnp.float32),
            pltpu.VMEM((C, H), jnp.float32),
            pltpu.VMEM((C, H), jnp.float32),
            pltpu.VMEM_SHARED((NPAD, H), jnp.float32),
            pltpu.SemaphoreType.DMA,
            pltpu.SemaphoreType.DMA,
            pltpu.SemaphoreType.DMA,
            pltpu.SemaphoreType.DMA,
            pltpu.SemaphoreType.DMA,
            pltpu.SemaphoreType.DMA,
            pltpu.SemaphoreType.DMA,
            pltpu.SemaphoreType.DMA,
            pltpu.SemaphoreType.DMA,
            pltpu.SemaphoreType.DMA,
        ],
    )(_agg_body)


def _agg_body(e_hbm, h_hbm, src_hbm, dst_hbm, zeros_hbm, out_hbm,
              src0, src1, dst0, dst1, e0, e1, g0, g1, agg_sh,
              ss0, ss1, sd0, sd1, se0, se1, sg0, sg1, sc0, sc1):
    c = lax.axis_index("c")
    s = lax.axis_index("s")
    wid = s * NC + c
    srcs, dsts, ebufs, gbufs = (src0, src1), (dst0, dst1), (e0, e1), (g0, g1)
    sss, sds, ses, sgs, scs = ((ss0, ss1), (sd0, sd1), (se0, se1),
                               (sg0, sg1), (sc0, sc1))
    base0 = wid * jnp.int32(EPT)
    bmax = base0 + jnp.int32(EPT - C)

    pltpu.sync_copy(zeros_hbm, agg_sh.at[pl.ds(s * (NPAD // NS), NPAD // NS)])
    plsc.subcore_barrier()

    def bs(x):
        return jnp.minimum(base0 + x * jnp.int32(C), bmax)

    def i_src(x, p):
        pltpu.async_copy(src_hbm.at[pl.ds(bs(x), C)], srcs[p], sss[p])

    def i_dst(x, p):
        pltpu.async_copy(dst_hbm.at[pl.ds(bs(x), C)], dsts[p], sds[p])

    def w_src(p):
        pltpu.make_async_copy(src_hbm.at[pl.ds(0, C)], srcs[p], sss[p]).wait()

    def w_dst(p):
        pltpu.make_async_copy(dst_hbm.at[pl.ds(0, C)], dsts[p], sds[p]).wait()

    def i_e(x, p):
        pltpu.async_copy(e_hbm.at[pl.ds(bs(x), C)], ebufs[p], ses[p])

    def w_e(p):
        pltpu.make_async_copy(e_hbm.at[pl.ds(0, C)], ebufs[p], ses[p]).wait()

    def i_g(p):
        pltpu.async_copy(h_hbm.at[srcs[p]], gbufs[p], sgs[p])

    def w_g(p):
        pltpu.make_async_copy(e_hbm.at[pl.ds(0, C)], gbufs[p], sgs[p]).wait()

    def i_sc(p):
        pltpu.async_copy(gbufs[p], agg_sh.at[dsts[p]], scs[p], add=True)

    def w_sc(p):
        pltpu.make_async_copy(g0, agg_sh.at[pl.ds(0, C)], scs[p]).wait()

    def compute(p):
        gb, eb = gbufs[p], ebufs[p]

        def rowfn(i, cr):
            for jj in range(H // LANES):
                sl = pl.ds(jj * LANES, LANES)
                gb[i, sl] = jnp.maximum(gb[i, sl] + eb[i, sl], 0.0)
            return cr

        lax.fori_loop(jnp.int32(0), jnp.int32(C), rowfn, jnp.int32(0))

    def phase(x, p, first=False, last=False):
        q = 1 - p
        w_src(q)
        if not first:
            w_sc(q)
        if not last:
            i_g(q)
            i_e(x + 1, q)
        w_e(p)
        w_g(p)
        if not last:
            i_src(x + 2, p)
            i_dst(x + 1, q)
        compute(p)
        w_dst(p)
        i_sc(p)

    i_src(jnp.int32(0), 0)
    i_src(jnp.int32(1), 1)
    i_dst(jnp.int32(0), 0)
    w_src(0)
    i_g(0)
    i_e(jnp.int32(0), 0)

    phase(jnp.int32(0), 0, first=True)
    phase(jnp.int32(1), 1)

    def body(k, cr):
        x = k * jnp.int32(2)
        phase(x, 0)
        phase(x + 1, 1)
        return cr

    lax.fori_loop(jnp.int32(1), jnp.int32(ITERS // 2), body, jnp.int32(0))

    phase(jnp.int32(ITERS - 1), 0, last=True)

    w_sc(0)
    plsc.subcore_barrier()

    row0 = s * (NPAD // NS)
    pltpu.sync_copy(agg_sh.at[pl.ds(row0, NPAD // NS)],
                    out_hbm.at[c, pl.ds(row0, NPAD // NS)])


def kernel(x, ei, ea, W_proj, b_proj, eps, We, be, Wm1, bm1, Wm2, bm2,
           gamma, beta):
    src = ei[0].astype(jnp.int32)
    dst = ei[1].astype(jnp.int32)
    zeros = jnp.zeros((NPAD // NS, H), jnp.float32)

    h = _proj(x, W_proj, b_proj.reshape(1, H), 1000)

    for l in range(L):
        e = _proj(ea, We[l], be[l].reshape(1, H), 4000)
        aggp = _build_agg()(e, h, src, dst, zeros)
        scale = (1.0 + eps[l]).reshape(1, 1).astype(jnp.float32)
        h = _node_update(h, aggp[0], aggp[1], scale, Wm1[l],
                         bm1[l].reshape(1, 2 * H), Wm2[l],
                         bm2[l].reshape(1, H), gamma[l].reshape(1, H),
                         beta[l].reshape(1, H), 1000)
    return h
